# Initial kernel scaffold; baseline (speedup 1.0000x reference)
import jax
import jax.numpy as jnp
from jax.experimental import pallas as pl


def kernel(x, edge_index, W1, b1, W2, b2):
    raise NotImplementedError("write your pallas kernel here")


# speedup vs baseline: 5.1931x; 5.1931x over previous
import functools

import jax
import jax.numpy as jnp
from jax import lax
from jax.experimental import pallas as pl
from jax.experimental.pallas import tpu as pltpu
from jax.experimental.pallas import tpu_sc as plsc

N = 10000
E = 320000
D = 128

NC = 2
NS = 16
LANES = 16

K = 128
NW = NC * NS
EPW = E // NW
EPW_PAD = 10240
NCHUNK = EPW_PAD // K

EPT = E // NS
EPT_PAD = 20480
NCHUNK_DEG = EPT_PAD // K
NP = 10240
ROWS_PT = NP // NS
ZCH = ROWS_PT // K


def _vector_mesh():
  return plsc.VectorSubcoreMesh(core_axis_name="c", subcore_axis_name="s",
                                num_cores=NC, num_subcores=NS)


def _deg_call(eidx, onesk, zrows):
  @functools.partial(
      pl.kernel,
      out_type=jax.ShapeDtypeStruct((2, NP, D), jnp.float32),
      mesh=_vector_mesh(),
      scratch_types=[
          pltpu.VMEM((K,), jnp.int32),
          pltpu.VMEM((K, D), jnp.float32),
          pltpu.VMEM_SHARED((NP, D), jnp.float32),
      ],
  )
  def deg_kernel(eidx_hbm, ones_hbm, z_hbm, deg_hbm, idx_v, ones_v, acc_sh):
    c = lax.axis_index("c")
    s = lax.axis_index("s")
    w = c * NS + s

    pltpu.sync_copy(ones_hbm, ones_v)
    pltpu.sync_copy(z_hbm, acc_sh.at[pl.ds(s * ROWS_PT, ROWS_PT)])
    plsc.subcore_barrier()

    @pl.loop(0, NCHUNK_DEG)
    def _(j):
      pltpu.sync_copy(eidx_hbm.at[pl.ds(w * EPT_PAD + j * K, K)], idx_v)
      pltpu.sync_copy(ones_v, acc_sh.at[idx_v], add=True)

    plsc.subcore_barrier()
    sl = pl.ds(s * ROWS_PT, ROWS_PT)
    pltpu.sync_copy(acc_sh.at[sl], deg_hbm.at[c].at[sl])

  return deg_kernel(eidx, onesk, zrows)


def _agg_call(h, srcr, dstr, zrows):
  @functools.partial(
      pl.kernel,
      out_type=jax.ShapeDtypeStruct((2, NP, D), jnp.float32),
      mesh=_vector_mesh(),
      scratch_types=[
          pltpu.VMEM((K,), jnp.int32),
          pltpu.VMEM((K,), jnp.int32),
          pltpu.VMEM((K, D), jnp.float32),
          pltpu.VMEM_SHARED((NP, D), jnp.float32),
      ],
  )
  def agg_kernel(h_hbm, src_hbm, dst_hbm, z_hbm, part_hbm,
                 sidx_v, didx_v, rows_v, acc_sh):
    c = lax.axis_index("c")
    s = lax.axis_index("s")
    wid = s * NC + c

    pltpu.sync_copy(z_hbm, acc_sh.at[pl.ds(s * ROWS_PT, ROWS_PT)])
    plsc.subcore_barrier()

    @pl.loop(0, NCHUNK)
    def _(j):
      base = wid * EPW_PAD + j * K
      pltpu.sync_copy(src_hbm.at[pl.ds(base, K)], sidx_v)
      pltpu.sync_copy(dst_hbm.at[pl.ds(base, K)], didx_v)
      pltpu.sync_copy(h_hbm.at[sidx_v], rows_v)
      pltpu.sync_copy(rows_v, acc_sh.at[didx_v], add=True)

    plsc.subcore_barrier()
    sl = pl.ds(s * ROWS_PT, ROWS_PT)
    pltpu.sync_copy(acc_sh.at[sl], part_hbm.at[c].at[sl])

  return agg_kernel(h, srcr, dstr, zrows)


def _norm_col(deg_slice):
  return lax.rsqrt(jnp.maximum(deg_slice[0:N, 0:1], 1.0))


def _tc_prescale(x, deg):
  def body(x_ref, deg_ref, o_ref):
    o_ref[...] = x_ref[...] * _norm_col(deg_ref[0])

  return pl.pallas_call(
      body, out_shape=jax.ShapeDtypeStruct((N, D), jnp.float32))(x, deg)


def _tc_layer(parts, deg, W, b, relu_prescale):
  def body(p_ref, deg_ref, w_ref, b_ref, o_ref):
    agg = p_ref[0, 0:N, :] + p_ref[1, 0:N, :]
    nd = _norm_col(deg_ref[1])
    h = jnp.dot(agg * nd, w_ref[...],
                preferred_element_type=jnp.float32,
                precision=lax.Precision.HIGHEST)
    h = h + b_ref[...]
    if relu_prescale:
      h = jnp.maximum(h, 0.0) * _norm_col(deg_ref[0])
    o_ref[...] = h

  return pl.pallas_call(
      body, out_shape=jax.ShapeDtypeStruct((N, D), jnp.float32))(
          parts, deg, W, b)


def _pad_shard(arr, nshard, pad_vals):
  per = E // nshard
  sh = arr.reshape(nshard, per)
  pads = jnp.broadcast_to(pad_vals, (nshard, pad_vals.shape[0]))
  return jnp.concatenate([sh, pads], axis=1)


def kernel(x, edge_index, W1, b1, W2, b2):
  src_flat = edge_index[0]
  dst_flat = edge_index[1]

  pad_dummy_w = N + (jnp.arange(EPW_PAD - EPW, dtype=jnp.int32) % (NP - N))
  pad_dummy_t = N + (jnp.arange(EPT_PAD - EPT, dtype=jnp.int32) % (NP - N))
  pad_valid_w = (jnp.arange(EPW_PAD - EPW, dtype=jnp.int32) * 37) % N

  src = _pad_shard(src_flat, NW, pad_valid_w).reshape(NW * EPW_PAD)
  dst = _pad_shard(dst_flat, NW, pad_dummy_w).reshape(NW * EPW_PAD)
  eidx = jnp.concatenate([
      _pad_shard(src_flat, NS, pad_dummy_t),
      _pad_shard(dst_flat, NS, pad_dummy_t),
  ], axis=0).reshape(2 * NS * EPT_PAD)

  zrows = jnp.zeros((ROWS_PT, D), jnp.float32)
  onesk = jnp.ones((K, D), jnp.float32)
  b1r = b1.reshape(1, D)
  b2r = b2.reshape(1, D)

  deg = _deg_call(eidx, onesk, zrows)
  h0 = _tc_prescale(x, deg)
  p1 = _agg_call(h0, src, dst, zrows)
  h1 = _tc_layer(p1, deg, W1, b1r, relu_prescale=True)
  p2 = _agg_call(h1, src, dst, zrows)
  out = _tc_layer(p2, deg, W2, b2r, relu_prescale=False)
  return out
